# baseline (speedup 1.0000x reference)
import functools

import jax
import jax.numpy as jnp
from jax import lax
from jax.experimental import pallas as pl
from jax.experimental.pallas import tpu as pltpu
from jax.experimental.pallas import tpu_sc as plsc

N_NODES = 10000
N_EDGES = 320000
D_EDGE = 16
D_NODE = 128
D_OUT = 128

NC = 2
NS = 16
NW = NC * NS
BLK = 2560
N_BLK_REAL = N_EDGES // BLK
N_BLK_PAD = 4 * NW
N_PAD = N_BLK_PAD * BLK - N_EDGES
N_ACC = N_NODES + 1024


def _sc_scatter(tgt, rowsT, z1):
  mesh = plsc.VectorSubcoreMesh(core_axis_name="c", subcore_axis_name="s")

  @functools.partial(
      pl.kernel,
      out_type=(
          jax.ShapeDtypeStruct((NC, D_EDGE, N_ACC), jnp.float32),
          jax.ShapeDtypeStruct((NC, N_ACC), jnp.float32),
      ),
      mesh=mesh,
      scratch_types=(
          [pltpu.VMEM((BLK,), jnp.int32) for _ in range(2)],
          [[pltpu.VMEM((BLK,), jnp.float32) for _ in range(D_EDGE)]
           for _ in range(2)],
          pltpu.VMEM((BLK,), jnp.float32),
          [pltpu.VMEM_SHARED((N_ACC,), jnp.float32) for _ in range(D_EDGE)],
          pltpu.VMEM_SHARED((N_ACC,), jnp.float32),
          [pltpu.SemaphoreType.DMA for _ in range(2)],
          pltpu.SemaphoreType.DMA,
      ),
  )
  def k(idx_hbm, rowsT_hbm, z1_hbm, sums_out, cnt_out,
        idx_v, rowsT_v, ones_v, accs, cnt, sem_in, sem_sc):
    c = lax.axis_index("c")
    s = lax.axis_index("s")
    w = s * NC + c

    @pl.when(s == 0)
    def _zero():
      for j in range(D_EDGE):
        pltpu.sync_copy(z1_hbm, accs[j])
      pltpu.sync_copy(z1_hbm, cnt)

    for i in range(BLK // 16):
      ones_v[pl.ds(i * 16, 16)] = jnp.ones((16,), jnp.float32)

    plsc.subcore_barrier()

    def start_in(kb):
      b = kb % 2
      g = w + NW * kb
      base = pl.multiple_of(jnp.minimum(g, N_BLK_REAL - 1) * BLK, 128)
      descs = [pltpu.async_copy(idx_hbm.at[g], idx_v[b], sem_in[b])]
      descs += [
          pltpu.async_copy(rowsT_hbm.at[j, pl.ds(base, BLK)], rowsT_v[b][j],
                           sem_in[b])
          for j in range(D_EDGE)
      ]
      return descs

    def issue_scatters(b):
      descs = [pltpu.async_copy(rowsT_v[b][j], accs[j].at[idx_v[b]],
                                sem_sc, add=True) for j in range(D_EDGE)]
      descs.append(pltpu.async_copy(ones_v, cnt.at[idx_v[b]], sem_sc,
                                    add=True))
      return descs

    n_blk = N_BLK_PAD // NW
    in_descs = start_in(0)
    sc_descs = None
    for kb in range(n_blk):
      b = kb % 2
      if sc_descs is not None:
        for d in sc_descs:
          d.wait()
      if kb + 1 < n_blk:
        next_in = start_in(kb + 1)
      else:
        next_in = None
      for d in in_descs:
        d.wait()
      sc_descs = issue_scatters(b)
      in_descs = next_in
    for d in sc_descs:
      d.wait()

    plsc.subcore_barrier()

    @pl.when(s == 0)
    def _flush():
      for j in range(D_EDGE):
        pltpu.sync_copy(accs[j], sums_out.at[c, j])
      pltpu.sync_copy(cnt, cnt_out.at[c])

  return k(tgt, rowsT, z1)


def _tc_mm1_body(na_ref, w1_ref, b_ref, o_ref):
  acc = jnp.dot(na_ref[...], w1_ref[...], preferred_element_type=jnp.float32)
  o_ref[...] = acc + b_ref[...]


def _tc_mm1(node_attr, W1, b2):
  bn = 2000
  return pl.pallas_call(
      _tc_mm1_body,
      grid=(N_NODES // bn,),
      in_specs=[
          pl.BlockSpec((bn, D_NODE), lambda i: (i, 0)),
          pl.BlockSpec((D_NODE, D_OUT), lambda i: (0, 0)),
          pl.BlockSpec((1, D_OUT), lambda i: (0, 0)),
      ],
      out_specs=pl.BlockSpec((bn, D_OUT), lambda i: (i, 0)),
      out_shape=jax.ShapeDtypeStruct((N_NODES, D_OUT), jnp.float32),
  )(node_attr, W1, b2)


def _tc_body(ps_ref, pc_ref, a1_ref, w2_ref, o_ref):
  sums = ps_ref[0] + ps_ref[1]
  cnt = pc_ref[0] + pc_ref[1]
  mean = sums * (1.0 / jnp.maximum(cnt, 1.0))
  acc = a1_ref[...]
  acc += jnp.dot(mean, w2_ref[...], preferred_element_type=jnp.float32)
  o_ref[...] = jnp.maximum(acc, 0.0)


def _tc_fused(psums, pcnt3, acc1, W2):
  bn = 2000
  grid = (N_NODES // bn,)
  return pl.pallas_call(
      _tc_body,
      grid=grid,
      in_specs=[
          pl.BlockSpec((NC, bn, D_EDGE), lambda i: (0, i, 0)),
          pl.BlockSpec((NC, bn, 1), lambda i: (0, i, 0)),
          pl.BlockSpec((bn, D_OUT), lambda i: (i, 0)),
          pl.BlockSpec((D_EDGE, D_OUT), lambda i: (0, 0)),
      ],
      out_specs=pl.BlockSpec((bn, D_OUT), lambda i: (i, 0)),
      out_shape=jax.ShapeDtypeStruct((N_NODES, D_OUT), jnp.float32),
  )(psums, pcnt3, acc1, W2)


def kernel(edge_index, edge_attr, node_attr, W, b):
  pad = N_NODES + (jnp.arange(N_PAD, dtype=jnp.int32) % 1024)
  tgt = jnp.concatenate([edge_index[0], pad]).reshape(N_BLK_PAD, BLK)
  rowsT = edge_attr.T
  z1 = jnp.zeros((N_ACC,), jnp.float32)
  psums_t, pcnt = _sc_scatter(tgt, rowsT, z1)
  psums = psums_t[:, :, :N_NODES].transpose(0, 2, 1)
  pcnt = pcnt[:, :N_NODES]
  W1 = W[:D_NODE]
  W2 = W[D_NODE:]
  b2 = b.reshape(1, D_OUT)
  acc1 = _tc_mm1(node_attr, W1, b2)
  return _tc_fused(psums, pcnt[:, :, None], acc1, W2)


# speedup vs baseline: 1.0015x; 1.0015x over previous
import functools

import jax
import jax.numpy as jnp
from jax import lax
from jax.experimental import pallas as pl
from jax.experimental.pallas import tpu as pltpu
from jax.experimental.pallas import tpu_sc as plsc

N_NODES = 10000
N_EDGES = 320000
D_EDGE = 16
D_NODE = 128
D_OUT = 128

NC = 2
NS = 16
NW = NC * NS
BLK = 2560
N_BLK_REAL = N_EDGES // BLK
N_BLK_PAD = 4 * NW
N_PAD = N_BLK_PAD * BLK - N_EDGES
N_ACC = N_NODES + 128


def _sc_scatter(tgt, rowsT, z1):
  mesh = plsc.VectorSubcoreMesh(core_axis_name="c", subcore_axis_name="s")

  @functools.partial(
      pl.kernel,
      out_type=(
          jax.ShapeDtypeStruct((NC, D_EDGE, N_ACC), jnp.float32),
          jax.ShapeDtypeStruct((NC, N_ACC), jnp.float32),
      ),
      mesh=mesh,
      scratch_types=(
          [pltpu.VMEM((BLK,), jnp.int32) for _ in range(2)],
          [[pltpu.VMEM((BLK,), jnp.float32) for _ in range(D_EDGE)]
           for _ in range(2)],
          pltpu.VMEM((BLK,), jnp.float32),
          [pltpu.VMEM_SHARED((N_ACC,), jnp.float32) for _ in range(D_EDGE)],
          pltpu.VMEM_SHARED((N_ACC,), jnp.float32),
          [pltpu.SemaphoreType.DMA for _ in range(2)],
          pltpu.SemaphoreType.DMA,
      ),
  )
  def k(idx_hbm, rowsT_hbm, z1_hbm, sums_out, cnt_out,
        idx_v, rowsT_v, ones_v, accs, cnt, sem_in, sem_sc):
    c = lax.axis_index("c")
    s = lax.axis_index("s")
    w = s * NC + c

    @pl.when(s == 0)
    def _zero():
      for j in range(D_EDGE):
        pltpu.sync_copy(z1_hbm, accs[j])
      pltpu.sync_copy(z1_hbm, cnt)

    for i in range(BLK // 16):
      ones_v[pl.ds(i * 16, 16)] = jnp.ones((16,), jnp.float32)

    plsc.subcore_barrier()

    def start_in(kb):
      b = kb % 2
      g = w + NW * kb
      base = pl.multiple_of(jnp.minimum(g, N_BLK_REAL - 1) * BLK, 128)
      descs = [pltpu.async_copy(idx_hbm.at[g], idx_v[b], sem_in[b])]
      descs += [
          pltpu.async_copy(rowsT_hbm.at[j, pl.ds(base, BLK)], rowsT_v[b][j],
                           sem_in[b])
          for j in range(D_EDGE)
      ]
      return descs

    def issue_scatters(b):
      descs = [pltpu.async_copy(rowsT_v[b][j], accs[j].at[idx_v[b]],
                                sem_sc, add=True) for j in range(D_EDGE)]
      descs.append(pltpu.async_copy(ones_v, cnt.at[idx_v[b]], sem_sc,
                                    add=True))
      return descs

    n_blk = N_BLK_PAD // NW
    in_descs = start_in(0)
    sc_descs = None
    for kb in range(n_blk):
      b = kb % 2
      if sc_descs is not None:
        for d in sc_descs:
          d.wait()
      if kb + 1 < n_blk:
        next_in = start_in(kb + 1)
      else:
        next_in = None
      for d in in_descs:
        d.wait()
      sc_descs = issue_scatters(b)
      in_descs = next_in
    for d in sc_descs:
      d.wait()

    plsc.subcore_barrier()

    @pl.when(s == 0)
    def _flush():
      for j in range(D_EDGE):
        pltpu.sync_copy(accs[j], sums_out.at[c, j])
      pltpu.sync_copy(cnt, cnt_out.at[c])

  return k(tgt, rowsT, z1)


def _tc_mm1_body(na_ref, w1_ref, b_ref, o_ref):
  acc = jnp.dot(na_ref[...], w1_ref[...], preferred_element_type=jnp.float32)
  o_ref[...] = acc + b_ref[...]


def _tc_mm1(node_attr, W1, b2):
  bn = 2000
  return pl.pallas_call(
      _tc_mm1_body,
      grid=(N_NODES // bn,),
      in_specs=[
          pl.BlockSpec((bn, D_NODE), lambda i: (i, 0)),
          pl.BlockSpec((D_NODE, D_OUT), lambda i: (0, 0)),
          pl.BlockSpec((1, D_OUT), lambda i: (0, 0)),
      ],
      out_specs=pl.BlockSpec((bn, D_OUT), lambda i: (i, 0)),
      out_shape=jax.ShapeDtypeStruct((N_NODES, D_OUT), jnp.float32),
  )(node_attr, W1, b2)


def _tc_body(ps_ref, pc_ref, a1_ref, w2_ref, o_ref):
  sums = ps_ref[0] + ps_ref[1]
  cnt = pc_ref[0] + pc_ref[1]
  mean = sums * (1.0 / jnp.maximum(cnt, 1.0))
  acc = a1_ref[...]
  acc += jnp.dot(mean, w2_ref[...], preferred_element_type=jnp.float32)
  o_ref[...] = jnp.maximum(acc, 0.0)


def _tc_fused(psums, pcnt3, acc1, W2):
  bn = 2000
  grid = (N_NODES // bn,)
  return pl.pallas_call(
      _tc_body,
      grid=grid,
      in_specs=[
          pl.BlockSpec((NC, bn, D_EDGE), lambda i: (0, i, 0)),
          pl.BlockSpec((NC, bn, 1), lambda i: (0, i, 0)),
          pl.BlockSpec((bn, D_OUT), lambda i: (i, 0)),
          pl.BlockSpec((D_EDGE, D_OUT), lambda i: (0, 0)),
      ],
      out_specs=pl.BlockSpec((bn, D_OUT), lambda i: (i, 0)),
      out_shape=jax.ShapeDtypeStruct((N_NODES, D_OUT), jnp.float32),
  )(psums, pcnt3, acc1, W2)


def kernel(edge_index, edge_attr, node_attr, W, b):
  pad = N_NODES + (jnp.arange(N_PAD, dtype=jnp.int32) % 128)
  tgt = jnp.concatenate([edge_index[0], pad]).reshape(N_BLK_PAD, BLK)
  rowsT = edge_attr.T
  z1 = jnp.zeros((N_ACC,), jnp.float32)
  psums_t, pcnt = _sc_scatter(tgt, rowsT, z1)
  psums = psums_t[:, :, :N_NODES].transpose(0, 2, 1)
  pcnt = pcnt[:, :N_NODES]
  W1 = W[:D_NODE]
  W2 = W[D_NODE:]
  b2 = b.reshape(1, D_OUT)
  acc1 = _tc_mm1(node_attr, W1, b2)
  return _tc_fused(psums, pcnt[:, :, None], acc1, W2)
